# Initial kernel scaffold; baseline (speedup 1.0000x reference)
import jax
import jax.numpy as jnp
from jax.experimental import pallas as pl


def kernel(item, out):
    raise NotImplementedError("write your pallas kernel here")


# speedup vs baseline: 4.9509x; 4.9509x over previous
import jax
import jax.numpy as jnp
from jax.experimental import pallas as pl


def _copy_body(src_ref, dst_ref):
    dst_ref[...] = src_ref[...]


def _pallas_row_copy(src, n_rows):
    dim = src.shape[1]
    block = min(n_rows, 2048)
    while n_rows % block:
        block //= 2
    grid = (n_rows // block,)
    return pl.pallas_call(
        _copy_body,
        grid=grid,
        in_specs=[pl.BlockSpec((block, dim), lambda i: (i, 0))],
        out_specs=pl.BlockSpec((block, dim), lambda i: (i, 0)),
        out_shape=jax.ShapeDtypeStruct((n_rows, dim), src.dtype),
    )(src[:n_rows])


def kernel(item, out):
    max_length = out.shape[0]
    b = item.shape[0]
    if max_length == 0:
        return item
    if b < max_length:
        return _pallas_row_copy(item, b)
    remaining = max_length
    filled = _pallas_row_copy(item, remaining)
    tail = item[remaining:]
    if tail.shape[0]:
        filled = jax.lax.dynamic_update_slice(filled, tail, (0, 0))
    return filled
